# Initial kernel scaffold; baseline (speedup 1.0000x reference)
import jax
import jax.numpy as jnp
from jax.experimental import pallas as pl


def kernel(x, enc_W0, enc_b0, enc_W1, enc_b1, enc_W2, enc_b2, enc_W3, enc_b3, dec_W0, dec_b0, dec_W1, dec_b1, dec_W2, dec_b2, dec_W3, dec_b3, codebook0, codebook1, codebook2, codebook3):
    raise NotImplementedError("write your pallas kernel here")


# speedup vs baseline: 1.8059x; 1.8059x over previous
import functools

import jax
import jax.numpy as jnp
from jax.experimental import pallas as pl

_B = 16384
_E = 64
_NCODE = 256
_BETA = 0.25
_BLK = 1024


def _dot(a, b):
    return jax.lax.dot_general(
        a, b, (((1,), (0,)), ((), ())), preferred_element_type=jnp.float32)


def _dot_t(a, b):
    return jax.lax.dot_general(
        a, b, (((1,), (1,)), ((), ())), preferred_element_type=jnp.float32)


def _dot_exact(a, b):
    return jax.lax.dot_general(
        a, b, (((1,), (0,)), ((), ())), precision=jax.lax.Precision.HIGHEST,
        preferred_element_type=jnp.float32)


def _body(x_ref,
          ew0, eb0, ew1, eb1, ew2, eb2, ew3, eb3,
          dw0, db0, dw1, db1, dw2, db2, dw3, db3,
          cb0, cb1, cb2, cb3,
          out_ref, loss_ref, idx_ref):
    i = pl.program_id(0)
    h = x_ref[...]
    h = jnp.maximum(_dot(h, ew0[...]) + eb0[...], 0.0)
    h = jnp.maximum(_dot(h, ew1[...]) + eb1[...], 0.0)
    h = jnp.maximum(_dot(h, ew2[...]) + eb2[...], 0.0)
    z = _dot(h, ew3[...]) + eb3[...]

    res = z
    xq = jnp.zeros_like(z)
    idx_acc = jnp.zeros((_BLK, 4), jnp.int32)
    lane4 = jax.lax.broadcasted_iota(jnp.int32, (_BLK, 4), 1)
    loss_vec = jnp.zeros((8, 128), jnp.float32)
    loss_rows = jax.lax.broadcasted_iota(jnp.int32, (8, 128), 0)

    for l, cb in enumerate((cb0, cb1, cb2, cb3)):
        c = cb[...]
        d = (jnp.sum(res * res, axis=1, keepdims=True)
             + jnp.sum(c * c, axis=1)[None, :]) - 2.0 * _dot_t(res, c)
        dmin = jnp.min(d, axis=1, keepdims=True)
        code_iota = jax.lax.broadcasted_iota(jnp.int32, d.shape, 1)
        idx = jnp.min(jnp.where(d <= dmin, code_iota, _NCODE),
                      axis=1, keepdims=True)
        onehot = (code_iota == idx).astype(jnp.float32)
        zq = _dot_exact(onehot, c)
        diff = zq - res
        s = jnp.sum(diff * diff)
        loss_vec = loss_vec + jnp.where(loss_rows == l, s, 0.0)
        xq = xq + zq
        res = res - zq
        idx_acc = jnp.where(lane4 == l, idx, idx_acc)

    g = jnp.maximum(_dot(xq, dw0[...]) + db0[...], 0.0)
    g = jnp.maximum(_dot(g, dw1[...]) + db1[...], 0.0)
    g = jnp.maximum(_dot(g, dw2[...]) + db2[...], 0.0)
    out_ref[...] = _dot(g, dw3[...]) + db3[...]

    idx_ref[...] = idx_acc

    @pl.when(i == 0)
    def _init():
        loss_ref[...] = jnp.zeros_like(loss_ref)

    loss_ref[...] += loss_vec


def kernel(x, enc_W0, enc_b0, enc_W1, enc_b1, enc_W2, enc_b2, enc_W3, enc_b3,
           dec_W0, dec_b0, dec_W1, dec_b1, dec_W2, dec_b2, dec_W3, dec_b3,
           codebook0, codebook1, codebook2, codebook3):
    in_dim = x.shape[1]
    grid = (_B // _BLK,)

    def _full(a):
        return pl.BlockSpec(a.shape, lambda i: (0,) * a.ndim)

    biases = [b.reshape(1, -1) for b in
              (enc_b0, enc_b1, enc_b2, enc_b3, dec_b0, dec_b1, dec_b2, dec_b3)]
    ws = (enc_W0, enc_W1, enc_W2, enc_W3, dec_W0, dec_W1, dec_W2, dec_W3)
    cbs = (codebook0, codebook1, codebook2, codebook3)

    in_specs = [pl.BlockSpec((_BLK, in_dim), lambda i: (i, 0))]
    operands = [x]
    for w, b in zip(ws[:4], biases[:4]):
        in_specs += [_full(w), _full(b)]
        operands += [w, b]
    for w, b in zip(ws[4:], biases[4:]):
        in_specs += [_full(w), _full(b)]
        operands += [w, b]
    for cb in cbs:
        in_specs.append(_full(cb))
        operands.append(cb)

    out, loss_mat, idx = pl.pallas_call(
        _body,
        grid=grid,
        in_specs=in_specs,
        out_specs=[
            pl.BlockSpec((_BLK, in_dim), lambda i: (i, 0)),
            pl.BlockSpec((8, 128), lambda i: (0, 0)),
            pl.BlockSpec((_BLK, 4), lambda i: (i, 0)),
        ],
        out_shape=[
            jax.ShapeDtypeStruct((_B, in_dim), jnp.float32),
            jax.ShapeDtypeStruct((8, 128), jnp.float32),
            jax.ShapeDtypeStruct((_B, 4), jnp.int32),
        ],
    )(*operands)

    sums = loss_mat[:4, 0]
    means = sums / (_B * _E)
    rq_loss = jnp.mean(_BETA * means + means)
    return (out, rq_loss, idx)


# speedup vs baseline: 2.5874x; 1.4328x over previous
import functools

import jax
import jax.numpy as jnp
from jax.experimental import pallas as pl

_B = 16384
_E = 64
_NCODE = 256
_BETA = 0.25
_BLK = 1024


def _dot(a, b):
    return jax.lax.dot_general(
        a, b, (((1,), (0,)), ((), ())), preferred_element_type=jnp.float32)


def _dot_t(a, b):
    return jax.lax.dot_general(
        a, b, (((1,), (1,)), ((), ())), preferred_element_type=jnp.float32)


def _split_f32(c):
    hi = jax.lax.bitcast_convert_type(
        jax.lax.bitcast_convert_type(c, jnp.uint32) & jnp.uint32(0xFFFFFF00),
        jnp.float32)
    return hi, c - hi


def _body(x_ref,
          ew0, eb0, ew1, eb1, ew2, eb2, ew3, eb3,
          dw0, db0, dw1, db1, dw2, db2, dw3, db3,
          cbh0, cbl0, cbh1, cbl1, cbh2, cbl2, cbh3, cbl3,
          out_ref, loss_ref, idx_ref):
    i = pl.program_id(0)
    h = x_ref[...]
    h = jnp.maximum(_dot(h, ew0[...]) + eb0[...], 0.0)
    h = jnp.maximum(_dot(h, ew1[...]) + eb1[...], 0.0)
    h = jnp.maximum(_dot(h, ew2[...]) + eb2[...], 0.0)
    z = _dot(h, ew3[...]) + eb3[...]

    res = z
    xq = jnp.zeros_like(z)
    idx_acc = jnp.zeros((_BLK, 4), jnp.int32)
    lane4 = jax.lax.broadcasted_iota(jnp.int32, (_BLK, 4), 1)
    loss_vec = jnp.zeros((8, 128), jnp.float32)
    loss_rows = jax.lax.broadcasted_iota(jnp.int32, (8, 128), 0)

    for l, (cbh, cbl) in enumerate(((cbh0, cbl0), (cbh1, cbl1),
                                    (cbh2, cbl2), (cbh3, cbl3))):
        ch = cbh[...]
        cl = cbl[...]
        c = ch + cl
        d = (jnp.sum(res * res, axis=1, keepdims=True)
             + jnp.sum(c * c, axis=1)[None, :]) - 2.0 * _dot_t(res, c)
        dmin = jnp.min(d, axis=1, keepdims=True)
        code_iota = jax.lax.broadcasted_iota(jnp.int32, d.shape, 1)
        idx = jnp.min(jnp.where(d <= dmin, code_iota, _NCODE),
                      axis=1, keepdims=True)
        onehot = (code_iota == idx).astype(jnp.float32)
        zq = _dot(onehot, ch) + _dot(onehot, cl)
        diff = zq - res
        s = jnp.sum(diff * diff)
        loss_vec = loss_vec + jnp.where(loss_rows == l, s, 0.0)
        xq = xq + zq
        res = res - zq
        idx_acc = jnp.where(lane4 == l, idx, idx_acc)

    g = jnp.maximum(_dot(xq, dw0[...]) + db0[...], 0.0)
    g = jnp.maximum(_dot(g, dw1[...]) + db1[...], 0.0)
    g = jnp.maximum(_dot(g, dw2[...]) + db2[...], 0.0)
    out_ref[...] = _dot(g, dw3[...]) + db3[...]

    idx_ref[...] = idx_acc

    @pl.when(i == 0)
    def _init():
        loss_ref[...] = jnp.zeros_like(loss_ref)

    loss_ref[...] += loss_vec


def kernel(x, enc_W0, enc_b0, enc_W1, enc_b1, enc_W2, enc_b2, enc_W3, enc_b3,
           dec_W0, dec_b0, dec_W1, dec_b1, dec_W2, dec_b2, dec_W3, dec_b3,
           codebook0, codebook1, codebook2, codebook3):
    in_dim = x.shape[1]
    grid = (_B // _BLK,)

    def _full(a):
        return pl.BlockSpec(a.shape, lambda i: (0,) * a.ndim)

    biases = [b.reshape(1, -1) for b in
              (enc_b0, enc_b1, enc_b2, enc_b3, dec_b0, dec_b1, dec_b2, dec_b3)]
    ws = (enc_W0, enc_W1, enc_W2, enc_W3, dec_W0, dec_W1, dec_W2, dec_W3)
    cbs = (codebook0, codebook1, codebook2, codebook3)

    in_specs = [pl.BlockSpec((_BLK, in_dim), lambda i: (i, 0))]
    operands = [x]
    for w, b in zip(ws[:4], biases[:4]):
        in_specs += [_full(w), _full(b)]
        operands += [w, b]
    for w, b in zip(ws[4:], biases[4:]):
        in_specs += [_full(w), _full(b)]
        operands += [w, b]
    for cb in cbs:
        hi, lo = _split_f32(cb)
        in_specs += [_full(hi), _full(lo)]
        operands += [hi, lo]

    out, loss_mat, idx = pl.pallas_call(
        _body,
        grid=grid,
        in_specs=in_specs,
        out_specs=[
            pl.BlockSpec((_BLK, in_dim), lambda i: (i, 0)),
            pl.BlockSpec((8, 128), lambda i: (0, 0)),
            pl.BlockSpec((_BLK, 4), lambda i: (i, 0)),
        ],
        out_shape=[
            jax.ShapeDtypeStruct((_B, in_dim), jnp.float32),
            jax.ShapeDtypeStruct((8, 128), jnp.float32),
            jax.ShapeDtypeStruct((_B, 4), jnp.int32),
        ],
    )(*operands)

    sums = loss_mat[:4, 0]
    means = sums / (_B * _E)
    rq_loss = jnp.mean(_BETA * means + means)
    return (out, rq_loss, idx)
